# baseline (speedup 1.0000x reference)
import jax
import jax.numpy as jnp
from jax import lax
from jax.experimental import pallas as pl
from jax.experimental.pallas import tpu as pltpu
from jax.experimental.pallas import tpu_sc as plsc

B = 16384
D = 24
DS = 25
DP = 128
NC = 2
NS = 16
NW = NC * NS
BPW = B // NW
L = 16
NCHUNK = 4
CROWS = BPW // NCHUNK
GPC = CROWS // L


def _sc_body(weeks_hbm, seasons_hbm, hol_hbm, w_hbm,
             wk_idx_hbm, dy_idx_hbm, hl_idx_hbm,
             out_hbm,
             weeks_v, seasons_v, hol_v,
             wsc_v, comb_v,
             wk_v, dy_v, hl_v, stage_v, w_v, sem):
    wid = lax.axis_index("s") * NC + lax.axis_index("c")
    base = wid * BPW

    cps = [
        pltpu.make_async_copy(weeks_hbm, weeks_v, sem),
        pltpu.make_async_copy(seasons_hbm, seasons_v, sem),
        pltpu.make_async_copy(hol_hbm, hol_v, sem),
        pltpu.make_async_copy(w_hbm, w_v.at[pl.ds(0, 3)], sem),
        pltpu.make_async_copy(wk_idx_hbm.at[pl.ds(base, BPW)], wk_v, sem),
        pltpu.make_async_copy(dy_idx_hbm.at[pl.ds(base, BPW)], dy_v, sem),
        pltpu.make_async_copy(hl_idx_hbm.at[pl.ds(base, BPW)], hl_v, sem),
    ]
    for cp in cps:
        cp.start()
    for cp in cps:
        cp.wait()

    wv = w_v[pl.ds(0, L)]
    w1 = jnp.full((L,), wv[0], jnp.float32)
    w2 = jnp.full((L,), wv[1], jnp.float32)
    w3 = jnp.full((L,), wv[2], jnp.float32)
    lane = lax.iota(jnp.int32, L)

    for r in range(53):
        for off in (0, 8):
            wsc_v[pl.ds(r * DS + off, L)] = w1 * weeks_v[r, pl.ds(off, L)]

    for s in range(7):
        for h in range(2):
            r = (s * 2 + h) * DS
            for off in (0, 8):
                comb_v[pl.ds(r + off, L)] = (
                    w2 * seasons_v[s, pl.ds(off, L)]
                    + w3 * hol_v[h, pl.ds(off, L)]
                )

    dcols = []
    for t in range(D):
        x = lane + t
        dcols.append(jnp.where(x >= D, x - D, x))

    def group(g, carry):
        b0 = g * L
        wk = wk_v[pl.ds(b0, L)] * DS
        cb = dy_v[pl.ds(b0, L)] * (2 * DS) + hl_v[pl.ds(b0, L)] * DS
        rows24 = (b0 + lane) * D
        ilp = 6
        for t in range(D // ilp):
            ts = tuple(t + k * (D // ilp) for k in range(ilp))
            ab = [(plsc.load_gather(wsc_v, [wk + dcols[tt]]),
                   plsc.load_gather(comb_v, [cb + dcols[tt]])) for tt in ts]
            for tt, (a, b) in zip(ts, ab):
                plsc.store_scatter(stage_v, [rows24 + dcols[tt]], a + b)
        return carry

    out_cps = []
    for c in range(NCHUNK):
        lax.fori_loop(c * GPC, (c + 1) * GPC, group, 0)
        cp = pltpu.make_async_copy(
            stage_v.at[pl.ds(c * CROWS * D, CROWS * D)],
            out_hbm.at[pl.ds((base + c * CROWS) * D, CROWS * D)],
            sem,
        )
        cp.start()
        out_cps.append(cp)
    for cp in out_cps:
        cp.wait()


def kernel(weeks, seasons, holidays_tab, w1, w2, w3, week_idx, day_idx, holiday_idx):
    w = jnp.stack([w1, w2, w3])
    mesh = plsc.VectorSubcoreMesh(core_axis_name="c", subcore_axis_name="s")
    f = pl.kernel(
        _sc_body,
        mesh=mesh,
        compiler_params=pltpu.CompilerParams(needs_layout_passes=False),
        out_type=jax.ShapeDtypeStruct((B * D,), jnp.float32),
        scratch_types=[
            pltpu.VMEM((53, D), jnp.float32),
            pltpu.VMEM((7, D), jnp.float32),
            pltpu.VMEM((2, D), jnp.float32),
            pltpu.VMEM((53 * DS + 8,), jnp.float32),
            pltpu.VMEM((14 * DS + 8,), jnp.float32),
            pltpu.VMEM((BPW,), jnp.int32),
            pltpu.VMEM((BPW,), jnp.int32),
            pltpu.VMEM((BPW,), jnp.int32),
            pltpu.VMEM((BPW * D,), jnp.float32),
            pltpu.VMEM((L,), jnp.float32),
            pltpu.SemaphoreType.DMA,
        ],
    )
    flat = f(weeks, seasons, holidays_tab, w,
             week_idx, day_idx, holiday_idx)
    return flat.reshape(B, D)


# speedup vs baseline: 1.2564x; 1.2564x over previous
import jax
import jax.numpy as jnp
from jax import lax
from jax.experimental import pallas as pl
from jax.experimental.pallas import tpu as pltpu
from jax.experimental.pallas import tpu_sc as plsc

B = 16384
D = 24
DS = 25
DP = 128
NC = 2
NS = 16
NW = NC * NS
BPW = B // NW
L = 16
NCHUNK = 4
CROWS = BPW // NCHUNK
GPC = CROWS // L


def _sc_body(weeks_hbm, seasons_hbm, hol_hbm, w_hbm,
             wk_idx_hbm, dy_idx_hbm, hl_idx_hbm,
             out_hbm,
             weeks_v, seasons_v, hol_v,
             wsc_v, comb_v,
             wk_v, dy_v, hl_v, stage_v, w_v, sem):
    wid = lax.axis_index("s") * NC + lax.axis_index("c")
    base = wid * BPW

    cps = [
        pltpu.make_async_copy(weeks_hbm, weeks_v, sem),
        pltpu.make_async_copy(seasons_hbm, seasons_v, sem),
        pltpu.make_async_copy(hol_hbm, hol_v, sem),
        pltpu.make_async_copy(w_hbm, w_v.at[pl.ds(0, 3)], sem),
        pltpu.make_async_copy(wk_idx_hbm.at[pl.ds(base, BPW)], wk_v, sem),
        pltpu.make_async_copy(dy_idx_hbm.at[pl.ds(base, BPW)], dy_v, sem),
        pltpu.make_async_copy(hl_idx_hbm.at[pl.ds(base, BPW)], hl_v, sem),
    ]
    for cp in cps:
        cp.start()
    for cp in cps:
        cp.wait()

    wv = w_v[pl.ds(0, L)]
    w1 = jnp.full((L,), wv[0], jnp.float32)
    w2 = jnp.full((L,), wv[1], jnp.float32)
    w3 = jnp.full((L,), wv[2], jnp.float32)
    lane = lax.iota(jnp.int32, L)

    for r in range(53):
        for off in (0, 8):
            wsc_v[pl.ds(r * DS + off, L)] = w1 * weeks_v[r, pl.ds(off, L)]

    for s in range(7):
        for h in range(2):
            r = (s * 2 + h) * DS
            for off in (0, 8):
                comb_v[pl.ds(r + off, L)] = (
                    w2 * seasons_v[s, pl.ds(off, L)]
                    + w3 * hol_v[h, pl.ds(off, L)]
                )

    dcols = []
    for t in range(D):
        x = lane + t
        dcols.append(jnp.where(x >= D, x - D, x))

    def group(g, carry):
        b0 = g * L
        wk = wk_v[pl.ds(b0, L)] * DS
        cb = dy_v[pl.ds(b0, L)] * (2 * DS) + hl_v[pl.ds(b0, L)] * DS
        rows = b0 + lane
        ilp = 6
        for t in range(D // ilp):
            ts = tuple(t + k * (D // ilp) for k in range(ilp))
            ab = [(plsc.load_gather(wsc_v, [wk + dcols[tt]]),
                   plsc.load_gather(comb_v, [cb + dcols[tt]])) for tt in ts]
            for tt, (a, b) in zip(ts, ab):
                plsc.store_scatter(stage_v, [rows, dcols[tt]], a + b)
        return carry

    out_cps = []
    for c in range(NCHUNK):
        lax.fori_loop(c * GPC, (c + 1) * GPC, group, 0)
        cp = pltpu.make_async_copy(
            stage_v.at[pl.ds(c * CROWS, CROWS)],
            out_hbm.at[pl.ds(base + c * CROWS, CROWS)],
            sem,
        )
        cp.start()
        out_cps.append(cp)
    for cp in out_cps:
        cp.wait()


def kernel(weeks, seasons, holidays_tab, w1, w2, w3, week_idx, day_idx, holiday_idx):
    w = jnp.stack([w1, w2, w3])
    mesh = plsc.VectorSubcoreMesh(core_axis_name="c", subcore_axis_name="s")
    f = pl.kernel(
        _sc_body,
        mesh=mesh,
        compiler_params=pltpu.CompilerParams(needs_layout_passes=False),
        out_type=jax.ShapeDtypeStruct((B, DP), jnp.float32),
        scratch_types=[
            pltpu.VMEM((53, D), jnp.float32),
            pltpu.VMEM((7, D), jnp.float32),
            pltpu.VMEM((2, D), jnp.float32),
            pltpu.VMEM((53 * DS + 8,), jnp.float32),
            pltpu.VMEM((14 * DS + 8,), jnp.float32),
            pltpu.VMEM((BPW,), jnp.int32),
            pltpu.VMEM((BPW,), jnp.int32),
            pltpu.VMEM((BPW,), jnp.int32),
            pltpu.VMEM((BPW, DP), jnp.float32),
            pltpu.VMEM((L,), jnp.float32),
            pltpu.SemaphoreType.DMA,
        ],
    )
    padded = f(weeks, seasons, holidays_tab, w,
               week_idx, day_idx, holiday_idx)
    return padded[:, :D]
